# baseline (speedup 1.0000x reference)
import functools

import jax
import jax.numpy as jnp
from jax import lax
from jax.experimental import pallas as pl
from jax.experimental.pallas import tpu as pltpu
from jax.experimental.pallas import tpu_sc as plsc

N = 10000
E = 320000
D = 128

NC = 2
NS = 16
NW = NC * NS

C = 128
NCH = 80
EP = NW * NCH * C

LPR = D // 16
RPT = 624
TAIL = N - NS * RPT

_mesh = plsc.VectorSubcoreMesh(core_axis_name="c", subcore_axis_name="s")


@functools.partial(
    pl.kernel,
    mesh=_mesh,
    out_type=jax.ShapeDtypeStruct((NC, N, D), jnp.float32),
    scratch_types=[
        pltpu.VMEM((3, C), jnp.int32),
        pltpu.VMEM((3, C), jnp.int32),
        pltpu.VMEM((3, C), jnp.int32),
        pltpu.VMEM((3, C), jnp.int32),
        pltpu.VMEM((C, D), jnp.float32),
        pltpu.VMEM((C, D), jnp.float32),
        pltpu.VMEM_SHARED((N, D), jnp.float32),
        pltpu.SemaphoreType.DMA,
        pltpu.SemaphoreType.DMA,
        pltpu.SemaphoreType.DMA,
        pltpu.SemaphoreType.DMA,
        pltpu.SemaphoreType.DMA,
        pltpu.SemaphoreType.DMA,
        pltpu.SemaphoreType.DMA,
        pltpu.SemaphoreType.DMA,
    ],
)
def _scatter_kernel(feat, packed, out, pk0, pk1, pk2, pk3, rbuf0, rbuf1,
                    acc, semi0, semi1, semi2, semi3, semg0, semg1,
                    sems0, sems1):
    pks = (pk0, pk1, pk2, pk3)
    semi = (semi0, semi1, semi2, semi3)
    rbufs = (rbuf0, rbuf1)
    semg = (semg0, semg1)
    sems = (sems0, sems1)
    c = lax.axis_index("c")
    s = lax.axis_index("s")
    wid = s * NC + c
    zero16 = jnp.zeros((16,), jnp.float32)
    rbuf = rbuf0

    def z_body(r, carry):
        for k in range(LPR):
            rbuf[r, pl.ds(k * 16, 16)] = zero16
        return carry
    lax.fori_loop(0, C, z_body, 0)
    base = s * RPT
    rem = RPT % C
    for t in range(RPT // C):
        pltpu.sync_copy(rbuf, acc.at[pl.ds(base + t * C, C)])
    pltpu.sync_copy(rbuf.at[pl.ds(0, rem)],
                    acc.at[pl.ds(base + (RPT // C) * C, rem)])

    @pl.when(s == NS - 1)
    def _zero_tail():
        pltpu.sync_copy(rbuf.at[pl.ds(0, TAIL)], acc.at[pl.ds(NS * RPT, TAIL)])
    plsc.subcore_barrier()

    def scale_chunk(q, buf):
        def mul_body(g, carry2):
            vv = lax.bitcast_convert_type(pks[q][2, pl.ds(g * 16, 16)],
                                          jnp.float32)
            for u in range(16):
                v = vv[u]
                e = g * 16 + u
                for k in range(LPR):
                    buf[e, pl.ds(k * 16, 16)] = buf[e, pl.ds(k * 16, 16)] * v
            return carry2
        lax.fori_loop(0, C // 16, mul_body, 0)

    pltpu.async_copy(packed.at[wid, 0], pks[0], semi[0])
    pltpu.async_copy(packed.at[wid, 1], pks[1], semi[1])
    pltpu.make_async_copy(packed.at[wid, 0], pks[0], semi[0]).wait()
    pltpu.async_copy(feat.at[pks[0].at[0]], rbufs[0], semg[0])

    def outer_body(o, carry):
        for q in range(4):
            j = o * 4 + q
            b = q % 2
            pltpu.make_async_copy(feat.at[pks[q].at[0]], rbufs[b],
                                  semg[b]).wait()

            @pl.when(j >= 1)
            def _wait_prev_scatter():
                pltpu.make_async_copy(rbufs[1 - b],
                                      acc.at[pks[(q - 1) % 4].at[1]],
                                      sems[1 - b]).wait()

            @pl.when(j + 1 < NCH)
            def _next_gather():
                pltpu.make_async_copy(packed.at[wid, j + 1],
                                      pks[(q + 1) % 4],
                                      semi[(q + 1) % 4]).wait()
                pltpu.async_copy(feat.at[pks[(q + 1) % 4].at[0]],
                                 rbufs[1 - b], semg[1 - b])

            scale_chunk(q, rbufs[b])

            pltpu.async_copy(rbufs[b], acc.at[pks[q].at[1]], sems[b],
                             add=True)

            @pl.when(j + 2 < NCH)
            def _prefetch_meta():
                pltpu.async_copy(packed.at[wid, j + 2], pks[(q + 2) % 4],
                                 semi[(q + 2) % 4])
        return carry
    lax.fori_loop(0, NCH // 4, outer_body, 0)
    blast = (NCH - 1) % 2
    pltpu.make_async_copy(rbufs[blast], acc.at[pks[(NCH - 1) % 4].at[1]],
                          sems[blast]).wait()
    plsc.subcore_barrier()

    nfull = RPT // C
    for t in range(nfull + 1):
        sz = C if t < nfull else rem
        r0 = base + t * C
        pltpu.sync_copy(acc.at[pl.ds(r0, sz)], rbuf.at[pl.ds(0, sz)])
        pltpu.sync_copy(rbuf.at[pl.ds(0, sz)], out.at[c, pl.ds(r0, sz)])

    @pl.when(s == NS - 1)
    def _write_tail():
        pltpu.sync_copy(acc.at[pl.ds(NS * RPT, TAIL)], rbuf.at[pl.ds(0, TAIL)])
        pltpu.sync_copy(rbuf.at[pl.ds(0, TAIL)], out.at[c, pl.ds(NS * RPT, TAIL)])


def _combine_body(p0, p1, f, o):
    o[...] = p0[0] + p1[0] + f[...]


_BLK = 1000


def _combine(partials, features):
    return pl.pallas_call(
        _combine_body,
        grid=(N // _BLK,),
        in_specs=[
            pl.BlockSpec((1, _BLK, D), lambda i: (0, i, 0)),
            pl.BlockSpec((1, _BLK, D), lambda i: (1, i, 0)),
            pl.BlockSpec((_BLK, D), lambda i: (i, 0)),
        ],
        out_specs=pl.BlockSpec((_BLK, D), lambda i: (i, 0)),
        out_shape=jax.ShapeDtypeStruct((N, D), jnp.float32),
    )(partials, partials, features)


def kernel(features, lap_indices, lap_values):
    pad = EP - E
    row = jnp.pad(lap_indices[0], (0, pad)).reshape(NW, NCH, C)
    col = jnp.pad(lap_indices[1], (0, pad)).reshape(NW, NCH, C)
    vbits = lax.bitcast_convert_type(
        jnp.pad(lap_values, (0, pad)), jnp.int32).reshape(NW, NCH, C)
    packed = jnp.stack([col, row, vbits], axis=2)
    partials = _scatter_kernel(features, packed)
    return _combine(partials, features)


# speedup vs baseline: 1.0158x; 1.0158x over previous
import functools

import jax
import jax.numpy as jnp
from jax import lax
from jax.experimental import pallas as pl
from jax.experimental.pallas import tpu as pltpu
from jax.experimental.pallas import tpu_sc as plsc

N = 10000
E = 320000
D = 128

NC = 2
NS = 16
NW = NC * NS

C = 64
R = 4
NCH = 160
EP = NW * NCH * C

LPR = D // 16
CB = 112
RPT = 624
TAIL = N - NS * RPT

_mesh = plsc.VectorSubcoreMesh(core_axis_name="c", subcore_axis_name="s")


@functools.partial(
    pl.kernel,
    mesh=_mesh,
    out_type=jax.ShapeDtypeStruct((NC, N, D), jnp.float32),
    scratch_types=[
        pltpu.VMEM((3, C), jnp.int32),
        pltpu.VMEM((3, C), jnp.int32),
        pltpu.VMEM((3, C), jnp.int32),
        pltpu.VMEM((3, C), jnp.int32),
        pltpu.VMEM((R, C), jnp.int32),
        pltpu.VMEM((C, D), jnp.float32),
        pltpu.VMEM((C, D), jnp.float32),
        pltpu.VMEM((C, D), jnp.float32),
        pltpu.VMEM((C, D), jnp.float32),
        pltpu.VMEM((CB, D), jnp.float32),
        pltpu.VMEM_SHARED((N, D), jnp.float32),
        pltpu.SemaphoreType.DMA,
        pltpu.SemaphoreType.DMA,
        pltpu.SemaphoreType.DMA,
        pltpu.SemaphoreType.DMA,
        pltpu.SemaphoreType.DMA,
        pltpu.SemaphoreType.DMA,
        pltpu.SemaphoreType.DMA,
        pltpu.SemaphoreType.DMA,
        pltpu.SemaphoreType.DMA,
        pltpu.SemaphoreType.DMA,
        pltpu.SemaphoreType.DMA,
        pltpu.SemaphoreType.DMA,
    ],
)
def _scatter_kernel(feat, packed, out, pk0, pk1, pk2, pk3, rowring,
                    rb0, rb1, rb2, rb3, sbuf, acc,
                    semi0, semi1, semi2, semi3,
                    semg0, semg1, semg2, semg3,
                    sems0, sems1, sems2, sems3):
    pks = (pk0, pk1, pk2, pk3)
    semi = (semi0, semi1, semi2, semi3)
    rbufs = (rb0, rb1, rb2, rb3)
    semg = (semg0, semg1, semg2, semg3)
    sems = (sems0, sems1, sems2, sems3)
    c = lax.axis_index("c")
    s = lax.axis_index("s")
    wid = s * NC + c
    zero16 = jnp.zeros((16,), jnp.float32)

    def z_body(r, carry):
        for k in range(LPR):
            sbuf[r, pl.ds(k * 16, 16)] = zero16
        return carry
    lax.fori_loop(0, CB, z_body, 0)
    base = s * RPT
    rem = RPT % CB
    for t in range(RPT // CB):
        pltpu.sync_copy(sbuf, acc.at[pl.ds(base + t * CB, CB)])
    pltpu.sync_copy(sbuf.at[pl.ds(0, rem)],
                    acc.at[pl.ds(base + (RPT // CB) * CB, rem)])

    @pl.when(s == NS - 1)
    def _zero_tail():
        pltpu.sync_copy(sbuf.at[pl.ds(0, TAIL)], acc.at[pl.ds(NS * RPT, TAIL)])
    plsc.subcore_barrier()

    def scale_chunk(q, buf):
        def mul_body(g, carry2):
            vv = lax.bitcast_convert_type(pks[q][2, pl.ds(g * 16, 16)],
                                          jnp.float32)
            for u in range(16):
                v = vv[u]
                e = g * 16 + u
                for k in range(LPR):
                    buf[e, pl.ds(k * 16, 16)] = buf[e, pl.ds(k * 16, 16)] * v
            return carry2
        lax.fori_loop(0, C // 16, mul_body, 0)

    for m in range(3):
        pltpu.async_copy(packed.at[wid, m], pks[m], semi[m])
    pltpu.make_async_copy(packed.at[wid, 0], pks[0], semi[0]).wait()
    pltpu.async_copy(feat.at[pks[0].at[0]], rbufs[0], semg[0])
    pltpu.make_async_copy(packed.at[wid, 1], pks[1], semi[1]).wait()
    pltpu.async_copy(feat.at[pks[1].at[0]], rbufs[1], semg[1])

    def outer_body(o, carry):
        for q in range(R):
            j = o * R + q
            pltpu.make_async_copy(feat.at[pks[q].at[0]], rbufs[q],
                                  semg[q]).wait()
            for g in range(C // 16):
                rowring[q, pl.ds(g * 16, 16)] = pks[q][1, pl.ds(g * 16, 16)]
            scale_chunk(q, rbufs[q])
            pltpu.async_copy(rbufs[q], acc.at[rowring.at[q]], sems[q],
                             add=True)

            @pl.when(j >= 2)
            def _wait_scatter():
                pltpu.make_async_copy(rbufs[(q + 2) % R],
                                      acc.at[rowring.at[(q + 2) % R]],
                                      sems[(q + 2) % R]).wait()

            @pl.when(j + 2 < NCH)
            def _next_gather():
                pltpu.make_async_copy(packed.at[wid, j + 2],
                                      pks[(q + 2) % R],
                                      semi[(q + 2) % R]).wait()
                pltpu.async_copy(feat.at[pks[(q + 2) % R].at[0]],
                                 rbufs[(q + 2) % R], semg[(q + 2) % R])

            @pl.when(j + 3 < NCH)
            def _prefetch_meta():
                pltpu.async_copy(packed.at[wid, j + 3], pks[(q + 3) % R],
                                 semi[(q + 3) % R])
        return carry
    lax.fori_loop(0, NCH // R, outer_body, 0)
    for j in (NCH - 2, NCH - 1):
        pltpu.make_async_copy(rbufs[j % R], acc.at[rowring.at[j % R]],
                              sems[j % R]).wait()
    plsc.subcore_barrier()

    nfull = RPT // CB
    for t in range(nfull + 1):
        sz = CB if t < nfull else rem
        r0 = base + t * CB
        pltpu.sync_copy(acc.at[pl.ds(r0, sz)], sbuf.at[pl.ds(0, sz)])
        pltpu.sync_copy(sbuf.at[pl.ds(0, sz)], out.at[c, pl.ds(r0, sz)])

    @pl.when(s == NS - 1)
    def _write_tail():
        pltpu.sync_copy(acc.at[pl.ds(NS * RPT, TAIL)], sbuf.at[pl.ds(0, TAIL)])
        pltpu.sync_copy(sbuf.at[pl.ds(0, TAIL)], out.at[c, pl.ds(NS * RPT, TAIL)])


def _combine_body(p0, p1, f, o):
    o[...] = p0[0] + p1[0] + f[...]


_BLK = 1000


def _combine(partials, features):
    return pl.pallas_call(
        _combine_body,
        grid=(N // _BLK,),
        in_specs=[
            pl.BlockSpec((1, _BLK, D), lambda i: (0, i, 0)),
            pl.BlockSpec((1, _BLK, D), lambda i: (1, i, 0)),
            pl.BlockSpec((_BLK, D), lambda i: (i, 0)),
        ],
        out_specs=pl.BlockSpec((_BLK, D), lambda i: (i, 0)),
        out_shape=jax.ShapeDtypeStruct((N, D), jnp.float32),
    )(partials, partials, features)


def kernel(features, lap_indices, lap_values):
    pad = EP - E
    row = jnp.pad(lap_indices[0], (0, pad)).reshape(NW, NCH, C)
    col = jnp.pad(lap_indices[1], (0, pad)).reshape(NW, NCH, C)
    vbits = lax.bitcast_convert_type(
        jnp.pad(lap_values, (0, pad)), jnp.int32).reshape(NW, NCH, C)
    packed = jnp.stack([col, row, vbits], axis=2)
    partials = _scatter_kernel(features, packed)
    return _combine(partials, features)
